# baseline (speedup 1.0000x reference)
import jax
import jax.numpy as jnp
from jax import lax
from jax.experimental import pallas as pl
from jax.experimental.pallas import tpu as pltpu
from jax.experimental.pallas import tpu_sc as plsc

VOCAB = 32000
D = 1024
E = 8
C = 640
F = 2048
T = 2048
NH = 2
EC = E * C

NC, NS, L = 2, 16, 16
NW = NC * NS
ROWS_PW = T // NW
SLOTS_PW = EC // NW
GCH = 80

BF = jnp.bfloat16
F32 = jnp.float32
I32 = jnp.int32

_mesh = plsc.VectorSubcoreMesh(core_axis_name="c", subcore_axis_name="s")
_sc_params = pltpu.CompilerParams(needs_layout_passes=False)


def _wid():
    return lax.axis_index("s") * NC + lax.axis_index("c")


def _embed_body(tab, ids, out, idx_v, rows_v, sem):
    base = _wid() * ROWS_PW
    pltpu.sync_copy(ids.at[pl.ds(base, ROWS_PW)], idx_v)
    pltpu.async_copy(tab.at[idx_v], rows_v, sem).wait()
    pltpu.sync_copy(rows_v, out.at[pl.ds(base, ROWS_PW)])


_embed_gather = pl.kernel(
    _embed_body,
    compiler_params=_sc_params,
    out_type=jax.ShapeDtypeStruct((T, D), F32),
    mesh=_mesh,
    scratch_types=[
        pltpu.VMEM((ROWS_PW,), I32),
        pltpu.VMEM((ROWS_PW, D), F32),
        pltpu.SemaphoreType.DMA,
    ],
)


def _router_math(h, rw, rb, d1r, d2r, c1r, c2r, w1r, w2r, rhor, hbf_r):
    hbf_r[...] = h.astype(BF)
    lg = lax.dot_general(h.astype(BF), rw, (((1,), (0,)), ((), ())),
                         preferred_element_type=F32)
    lane = lax.broadcasted_iota(I32, (T, 128), 1)
    real = lane < E
    lg = jnp.where(real, lg + rb, -1e30)
    m = jnp.max(lg, axis=1, keepdims=True)
    ex = jnp.where(real, jnp.exp(lg - m), 0.0)
    p = ex / jnp.sum(ex, axis=1, keepdims=True)
    m1 = jnp.max(p, axis=1, keepdims=True)
    i1 = jnp.min(jnp.where(p == m1, lane, 128), axis=1, keepdims=True)
    s1 = lane == i1
    p_x = jnp.where(s1, -1.0, p)
    m2 = jnp.max(p_x, axis=1, keepdims=True)
    i2 = jnp.min(jnp.where(p_x == m2, lane, 128), axis=1, keepdims=True)
    s2 = lane == i2
    maskf = jnp.where(s1 | s2, 1.0, 0.0)
    cs = maskf
    sh = 1
    while sh < T:
        cs = cs + jnp.concatenate([jnp.zeros((sh, 128), F32), cs[:T - sh]], axis=0)
        sh *= 2
    pos = cs - 1.0
    p1 = jnp.sum(jnp.where(s1, pos, 0.0), axis=1, keepdims=True)
    p2 = jnp.sum(jnp.where(s2, pos, 0.0), axis=1, keepdims=True)
    w1 = jnp.sum(jnp.where(s1, p, 0.0), axis=1, keepdims=True)
    w2 = jnp.sum(jnp.where(s2, p, 0.0), axis=1, keepdims=True)
    k1 = p1 < C
    k2 = p2 < C
    e1f = i1.astype(F32)
    e2f = i2.astype(F32)
    d1r[...] = jnp.where(k1, e1f * C + p1, float(EC)).astype(I32)
    d2r[...] = jnp.where(k2, e2f * C + p2, float(EC)).astype(I32)
    c1r[...] = jnp.where(k1, e1f * C + p1, 0.0).astype(I32)
    c2r[...] = jnp.where(k2, e2f * C + p2, 0.0).astype(I32)
    w1o = jnp.where(k1, w1, 0.0)
    w2o = jnp.where(k2, w2, 0.0)
    w1r[...] = w1o.astype(BF).astype(F32)
    w2r[...] = w2o.astype(BF).astype(F32)
    rhor[...] = w1o + w2o


def _router0_body(h_ref, rw_ref, rb_ref,
                  d1r, d2r, c1r, c2r, w1r, w2r, rhor, hbf_r):
    _router_math(h_ref[...], rw_ref[...], rb_ref[...],
                 d1r, d2r, c1r, c2r, w1r, w2r, rhor, hbf_r)


def _hop_update(h, r1, r2, w1b, w2b, rho):
    return (h + (w1b * r1.astype(F32) + w2b * r2.astype(F32))) - rho * h


def _router1_body(h_ref, r1_ref, r2_ref, wp1_ref, wp2_ref, rhop_ref, rw_ref, rb_ref,
                  d1r, d2r, c1r, c2r, w1r, w2r, rhor, hbf_r, hn_r):
    h = _hop_update(h_ref[...], r1_ref[...], r2_ref[...],
                    wp1_ref[...], wp2_ref[...], rhop_ref[...])
    hn_r[...] = h
    _router_math(h, rw_ref[...], rb_ref[...],
                 d1r, d2r, c1r, c2r, w1r, w2r, rhor, hbf_r)


_ROUTER_OUTS = ([jax.ShapeDtypeStruct((T, 1), I32)] * 4
                + [jax.ShapeDtypeStruct((T, 1), F32)] * 3
                + [jax.ShapeDtypeStruct((T, D), BF)])

_router0 = pl.pallas_call(_router0_body, out_shape=_ROUTER_OUTS)
_router1 = pl.pallas_call(
    _router1_body,
    out_shape=_ROUTER_OUTS + [jax.ShapeDtypeStruct((T, D), F32)],
)


def _disp_body(hbf_hbm, d1_hbm, d2_hbm, out_hbm, d1_v, d2_v, s2t_v, ra_v, rb_v, sem):
    pltpu.sync_copy(d1_hbm, d1_v)
    pltpu.sync_copy(d2_hbm, d2_v)

    def z_body(i, _):
        s2t_v[pl.ds(i * L, L)] = jnp.zeros((L,), I32)
        return 0

    lax.fori_loop(0, (EC + L) // L, z_body, 0, unroll=4)

    def sc_body(i, _):
        vals = lax.iota(I32, L) + i * L
        i1 = d1_v[pl.ds(i * L, L)]
        i2 = d2_v[pl.ds(i * L, L)]
        plsc.store_scatter(s2t_v, [i1], vals, mask=i1 < EC)
        plsc.store_scatter(s2t_v, [i2], vals, mask=i2 < EC)
        return 0

    lax.fori_loop(0, T // L, sc_body, 0, unroll=4)
    seg = _wid() * SLOTS_PW
    cpa = pltpu.async_copy(hbf_hbm.at[s2t_v.at[pl.ds(seg, GCH)]], ra_v, sem)
    cpb = pltpu.async_copy(hbf_hbm.at[s2t_v.at[pl.ds(seg + GCH, GCH)]], rb_v, sem)
    cpa.wait()
    cpb.wait()
    pltpu.sync_copy(ra_v, out_hbm.at[pl.ds(seg, GCH)])
    pltpu.sync_copy(rb_v, out_hbm.at[pl.ds(seg + GCH, GCH)])


_dispatch = pl.kernel(
    _disp_body,
    compiler_params=_sc_params,
    out_type=jax.ShapeDtypeStruct((EC, D // 2), I32),
    mesh=_mesh,
    scratch_types=[
        pltpu.VMEM((T,), I32),
        pltpu.VMEM((T,), I32),
        pltpu.VMEM((EC + L,), I32),
        pltpu.VMEM((GCH, D // 2), I32),
        pltpu.VMEM((GCH, D // 2), I32),
        pltpu.SemaphoreType.DMA,
    ],
)


def _ffn_body(x_ref, w1_ref, b1_ref, w2_ref, b2_ref, o_ref):
    h1 = lax.dot_general(x_ref[0], w1_ref[0], (((1,), (0,)), ((), ())),
                         preferred_element_type=F32)
    h1 = jnp.maximum(h1 + b1_ref[0], 0.0)
    o = lax.dot_general(h1.astype(BF), w2_ref[0], (((1,), (0,)), ((), ())),
                        preferred_element_type=F32)
    o_ref[0] = (o + b2_ref[0]).astype(BF)


_ffn = pl.pallas_call(
    _ffn_body,
    grid=(E,),
    in_specs=[
        pl.BlockSpec((1, C, D), lambda e: (e, 0, 0)),
        pl.BlockSpec((1, D, F), lambda e: (e, 0, 0)),
        pl.BlockSpec((1, 1, F), lambda e: (e, 0, 0)),
        pl.BlockSpec((1, F, D), lambda e: (e, 0, 0)),
        pl.BlockSpec((1, 1, D), lambda e: (e, 0, 0)),
    ],
    out_specs=pl.BlockSpec((1, C, D), lambda e: (e, 0, 0)),
    out_shape=jax.ShapeDtypeStruct((E, C, D), BF),
)


def _gath_body(eo_hbm, c1_hbm, c2_hbm, r1_hbm, r2_hbm, c1_v, c2_v, r1_v, r2_v, sem):
    t0 = _wid() * ROWS_PW
    pltpu.sync_copy(c1_hbm.at[pl.ds(t0, ROWS_PW)], c1_v)
    pltpu.sync_copy(c2_hbm.at[pl.ds(t0, ROWS_PW)], c2_v)
    cpa = pltpu.async_copy(eo_hbm.at[c1_v], r1_v, sem)
    cpb = pltpu.async_copy(eo_hbm.at[c2_v], r2_v, sem)
    cpa.wait()
    cpb.wait()
    pltpu.sync_copy(r1_v, r1_hbm.at[pl.ds(t0, ROWS_PW)])
    pltpu.sync_copy(r2_v, r2_hbm.at[pl.ds(t0, ROWS_PW)])


_comb_gather = pl.kernel(
    _gath_body,
    compiler_params=_sc_params,
    out_type=[jax.ShapeDtypeStruct((T, D // 2), I32)] * 2,
    mesh=_mesh,
    scratch_types=[
        pltpu.VMEM((ROWS_PW,), I32),
        pltpu.VMEM((ROWS_PW,), I32),
        pltpu.VMEM((ROWS_PW, D // 2), I32),
        pltpu.VMEM((ROWS_PW, D // 2), I32),
        pltpu.SemaphoreType.DMA,
    ],
)


NB = 50
NBLK = VOCAB // NB

def _final_body(h_ref, r1_ref, r2_ref, w1_ref, w2_ref, rho_ref, ln_ref, w_ref,
                o_ref, nrm_ref):
    @pl.when(pl.program_id(0) == 0)
    def _():
        h = _hop_update(h_ref[...], r1_ref[...], r2_ref[...],
                        w1_ref[...], w2_ref[...], rho_ref[...])
        mean = jnp.mean(h * h, axis=1, keepdims=True)
        nrm_ref[...] = (h * lax.rsqrt(mean + 1e-6) * ln_ref[...]).astype(BF)

    o_ref[...] = lax.dot_general(nrm_ref[...], w_ref[...], (((1,), (1,)), ((), ())),
                                 preferred_element_type=F32)


_final = pl.pallas_call(
    _final_body,
    grid=(NB,),
    in_specs=[
        pl.BlockSpec((T, D), lambda j: (0, 0)),
        pl.BlockSpec((T, D), lambda j: (0, 0)),
        pl.BlockSpec((T, D), lambda j: (0, 0)),
        pl.BlockSpec((T, 1), lambda j: (0, 0)),
        pl.BlockSpec((T, 1), lambda j: (0, 0)),
        pl.BlockSpec((T, 1), lambda j: (0, 0)),
        pl.BlockSpec((1, D), lambda j: (0, 0)),
        pl.BlockSpec((NBLK, D), lambda j: (j, 0)),
    ],
    out_specs=pl.BlockSpec((T, NBLK), lambda j: (0, j)),
    out_shape=jax.ShapeDtypeStruct((T, VOCAB), F32),
    scratch_shapes=[pltpu.VMEM((T, D), BF)],
)


def kernel(ids_t, embed_W, ln_scale, router_W, router_b, W1, b1, W2, b2):
    ids = ids_t.astype(I32)
    h = _embed_gather(embed_W, ids)
    rw = jnp.pad(jnp.transpose(router_W, (0, 2, 1)).astype(BF),
                 ((0, 0), (0, 0), (0, 128 - E)))
    rbp = jnp.pad(router_b, ((0, 0), (0, 128 - E)))[:, None, :]
    W1b = W1.astype(BF)
    W2b = W2.astype(BF)

    def _to_i32(x):
        return lax.bitcast_convert_type(x.reshape(x.shape[0], D // 2, 2), I32)

    def _to_bf(x):
        return lax.bitcast_convert_type(x, BF).reshape(x.shape[0], D)

    def hop_ffn(hop_idx, d1, d2, c1, c2, hbf):
        exp_in = _dispatch(_to_i32(hbf), d1.reshape(T), d2.reshape(T))
        eo = _ffn(_to_bf(exp_in).reshape(E, C, D), W1b[hop_idx],
                  b1[hop_idx].reshape(E, 1, F), W2b[hop_idx],
                  b2[hop_idx].reshape(E, 1, D))
        r1i, r2i = _comb_gather(_to_i32(eo.reshape(EC, D)),
                                c1.reshape(T), c2.reshape(T))
        return _to_bf(r1i), _to_bf(r2i)

    d1, d2, c1, c2, w1b, w2b, rho, hbf = _router0(h, rw[0], rbp[0])
    ra, rb2 = hop_ffn(0, d1, d2, c1, c2, hbf)
    d1, d2, c1, c2, w1b2, w2b2, rho2, hbf, hn = _router1(
        h, ra, rb2, w1b, w2b, rho, rw[1], rbp[1])
    ra, rb2 = hop_ffn(1, d1, d2, c1, c2, hbf)
    return _final(hn, ra, rb2, w1b2, w2b2, rho2, ln_scale[None],
                  embed_W.astype(BF))


# speedup vs baseline: 1.7188x; 1.7188x over previous
import jax
import jax.numpy as jnp
from jax import lax
from jax.experimental import pallas as pl
from jax.experimental.pallas import tpu as pltpu
from jax.experimental.pallas import tpu_sc as plsc

VOCAB = 32000
D = 1024
E = 8
C = 640
F = 2048
T = 2048
NH = 2
EC = E * C

NC, NS, L = 2, 16, 16
NW = NC * NS
ROWS_PW = T // NW
SLOTS_PW = EC // NW
GCH = 40

BF = jnp.bfloat16
F32 = jnp.float32
I32 = jnp.int32

_mesh = plsc.VectorSubcoreMesh(core_axis_name="c", subcore_axis_name="s")
_sc_params = pltpu.CompilerParams(needs_layout_passes=False)


def _wid():
    return lax.axis_index("s") * NC + lax.axis_index("c")


def _embed_body(tab, ids, out, idx_v, rows_v, sem):
    base = _wid() * ROWS_PW
    pltpu.sync_copy(ids.at[pl.ds(base, ROWS_PW)], idx_v)
    pltpu.async_copy(tab.at[idx_v], rows_v, sem).wait()
    pltpu.sync_copy(rows_v, out.at[pl.ds(base, ROWS_PW)])


_embed_gather = pl.kernel(
    _embed_body,
    compiler_params=_sc_params,
    out_type=jax.ShapeDtypeStruct((T, D), F32),
    mesh=_mesh,
    scratch_types=[
        pltpu.VMEM((ROWS_PW,), I32),
        pltpu.VMEM((ROWS_PW, D), F32),
        pltpu.SemaphoreType.DMA,
    ],
)


def _router_math(h, rw, rb, d1r, d2r, c1r, c2r, w1r, w2r, rhor):
    lg = lax.dot_general(h.astype(BF), rw, (((1,), (0,)), ((), ())),
                         preferred_element_type=F32)
    lane = lax.broadcasted_iota(I32, (T, 128), 1)
    real = lane < E
    lg = jnp.where(real, lg + rb, -1e30)
    m = jnp.max(lg, axis=1, keepdims=True)
    ex = jnp.where(real, jnp.exp(lg - m), 0.0)
    p = ex / jnp.sum(ex, axis=1, keepdims=True)
    m1 = jnp.max(p, axis=1, keepdims=True)
    i1 = jnp.min(jnp.where(p == m1, lane, 128), axis=1, keepdims=True)
    s1 = lane == i1
    p_x = jnp.where(s1, -1.0, p)
    m2 = jnp.max(p_x, axis=1, keepdims=True)
    i2 = jnp.min(jnp.where(p_x == m2, lane, 128), axis=1, keepdims=True)
    s2 = lane == i2
    maskf = jnp.where(s1 | s2, 1.0, 0.0)
    cs = maskf
    sh = 1
    while sh < T:
        cs = cs + jnp.concatenate([jnp.zeros((sh, 128), F32), cs[:T - sh]], axis=0)
        sh *= 2
    pos = cs - 1.0
    p1 = jnp.sum(jnp.where(s1, pos, 0.0), axis=1, keepdims=True)
    p2 = jnp.sum(jnp.where(s2, pos, 0.0), axis=1, keepdims=True)
    w1 = jnp.sum(jnp.where(s1, p, 0.0), axis=1, keepdims=True)
    w2 = jnp.sum(jnp.where(s2, p, 0.0), axis=1, keepdims=True)
    k1 = p1 < C
    k2 = p2 < C
    e1f = i1.astype(F32)
    e2f = i2.astype(F32)
    d1r[...] = jnp.where(k1, e1f * C + p1, float(EC)).astype(I32)
    d2r[...] = jnp.where(k2, e2f * C + p2, float(EC)).astype(I32)
    c1r[...] = jnp.where(k1, e1f * C + p1, 0.0).astype(I32)
    c2r[...] = jnp.where(k2, e2f * C + p2, 0.0).astype(I32)
    w1o = jnp.where(k1, w1, 0.0)
    w2o = jnp.where(k2, w2, 0.0)
    w1r[...] = w1o.astype(BF).astype(F32)
    w2r[...] = w2o.astype(BF).astype(F32)
    rhor[...] = w1o + w2o


def _router0_body(h_ref, rw_ref, rb_ref,
                  d1r, d2r, c1r, c2r, w1r, w2r, rhor):
    _router_math(h_ref[...], rw_ref[...], rb_ref[...],
                 d1r, d2r, c1r, c2r, w1r, w2r, rhor)


def _hop_update(h, r1, r2, w1b, w2b, rho):
    return (h + (w1b * r1.astype(F32) + w2b * r2.astype(F32))) - rho * h


def _router1_body(h_ref, r1_ref, r2_ref, wp1_ref, wp2_ref, rhop_ref, rw_ref, rb_ref,
                  d1r, d2r, c1r, c2r, w1r, w2r, rhor, hn_r):
    h = _hop_update(h_ref[...], r1_ref[...], r2_ref[...],
                    wp1_ref[...], wp2_ref[...], rhop_ref[...])
    hn_r[...] = h
    _router_math(h, rw_ref[...], rb_ref[...],
                 d1r, d2r, c1r, c2r, w1r, w2r, rhor)


_ROUTER_OUTS = ([jax.ShapeDtypeStruct((T, 1), I32)] * 4
                + [jax.ShapeDtypeStruct((T, 1), F32)] * 3)

_router0 = pl.pallas_call(_router0_body, out_shape=_ROUTER_OUTS)
_router1 = pl.pallas_call(
    _router1_body,
    out_shape=_ROUTER_OUTS + [jax.ShapeDtypeStruct((T, D), F32)],
)


def _disp_body(h_hbm, d1_hbm, d2_hbm, out_hbm, d1_v, d2_v, s2t_v, ra_v, rb_v, sem):
    cp1 = pltpu.async_copy(d1_hbm, d1_v, sem)
    cp2 = pltpu.async_copy(d2_hbm, d2_v, sem)
    cp1.wait()
    cp2.wait()

    def z_body(i, _):
        s2t_v[pl.ds(i * L, L)] = jnp.zeros((L,), I32)
        return 0

    lax.fori_loop(0, (EC + L) // L, z_body, 0, unroll=4)

    def sc_body(i, _):
        vals = lax.iota(I32, L) + i * L
        i1 = d1_v[pl.ds(i * L, L)]
        i2 = d2_v[pl.ds(i * L, L)]
        plsc.store_scatter(s2t_v, [i1], vals, mask=i1 < EC)
        plsc.store_scatter(s2t_v, [i2], vals, mask=i2 < EC)
        return 0

    lax.fori_loop(0, T // L, sc_body, 0, unroll=4)
    seg = _wid() * SLOTS_PW
    bufs = [ra_v, rb_v]
    cps = [None, None]
    nch = SLOTS_PW // GCH
    for ch in range(nch):
        b = ch % 2
        if cps[b] is not None:
            cps[b].wait()
            pltpu.sync_copy(bufs[b], out_hbm.at[pl.ds(seg + (ch - 2) * GCH, GCH)])
        cps[b] = pltpu.async_copy(
            h_hbm.at[s2t_v.at[pl.ds(seg + ch * GCH, GCH)]], bufs[b], sem)
    for ch in (nch - 2, nch - 1):
        b = ch % 2
        cps[b].wait()
        pltpu.sync_copy(bufs[b], out_hbm.at[pl.ds(seg + ch * GCH, GCH)])


_dispatch = pl.kernel(
    _disp_body,
    compiler_params=_sc_params,
    out_type=jax.ShapeDtypeStruct((EC, D), F32),
    mesh=_mesh,
    scratch_types=[
        pltpu.VMEM((T,), I32),
        pltpu.VMEM((T,), I32),
        pltpu.VMEM((EC + L,), I32),
        pltpu.VMEM((GCH, D), F32),
        pltpu.VMEM((GCH, D), F32),
        pltpu.SemaphoreType.DMA,
    ],
)


def _ffn_body(x_ref, w1_ref, b1_ref, w2_ref, b2_ref, o_ref):
    h1 = lax.dot_general(x_ref[0].astype(BF), w1_ref[0], (((1,), (0,)), ((), ())),
                         preferred_element_type=F32)
    h1 = jnp.maximum(h1 + b1_ref[0], 0.0)
    o = lax.dot_general(h1.astype(BF), w2_ref[0], (((1,), (0,)), ((), ())),
                        preferred_element_type=F32)
    o_ref[0] = (o + b2_ref[0]).astype(BF).astype(F32)


_ffn = pl.pallas_call(
    _ffn_body,
    grid=(E,),
    in_specs=[
        pl.BlockSpec((1, C, D), lambda e: (e, 0, 0)),
        pl.BlockSpec((1, D, F), lambda e: (e, 0, 0)),
        pl.BlockSpec((1, 1, F), lambda e: (e, 0, 0)),
        pl.BlockSpec((1, F, D), lambda e: (e, 0, 0)),
        pl.BlockSpec((1, 1, D), lambda e: (e, 0, 0)),
    ],
    out_specs=pl.BlockSpec((1, C, D), lambda e: (e, 0, 0)),
    out_shape=jax.ShapeDtypeStruct((E, C, D), F32),
)


HGH = ROWS_PW // 2


def _gath_body(eo_hbm, c1_hbm, c2_hbm, r1_hbm, r2_hbm, c1_v, c2_v, ra_v, rb_v, sem):
    t0 = _wid() * ROWS_PW
    cp1 = pltpu.async_copy(c1_hbm.at[pl.ds(t0, ROWS_PW)], c1_v, sem)
    cp2 = pltpu.async_copy(c2_hbm.at[pl.ds(t0, ROWS_PW)], c2_v, sem)
    cp1.wait()
    cp2.wait()
    chunks = [(c1_v, r1_hbm, 0), (c2_v, r2_hbm, 0), (c1_v, r1_hbm, 1), (c2_v, r2_hbm, 1)]
    bufs = [ra_v, rb_v]
    cps = [None, None]
    for ch, (cv, out, half) in enumerate(chunks):
        b = ch % 2
        if cps[b] is not None:
            pcv, pout, phalf = chunks[ch - 2]
            cps[b].wait()
            pltpu.sync_copy(bufs[b], pout.at[pl.ds(t0 + phalf * HGH, HGH)])
        cps[b] = pltpu.async_copy(eo_hbm.at[cv.at[pl.ds(half * HGH, HGH)]], bufs[b], sem)
    for ch in (2, 3):
        b = ch % 2
        pcv, pout, phalf = chunks[ch]
        cps[b].wait()
        pltpu.sync_copy(bufs[b], pout.at[pl.ds(t0 + phalf * HGH, HGH)])


_comb_gather = pl.kernel(
    _gath_body,
    compiler_params=_sc_params,
    out_type=[jax.ShapeDtypeStruct((T, D), F32)] * 2,
    mesh=_mesh,
    scratch_types=[
        pltpu.VMEM((ROWS_PW,), I32),
        pltpu.VMEM((ROWS_PW,), I32),
        pltpu.VMEM((HGH, D), F32),
        pltpu.VMEM((HGH, D), F32),
        pltpu.SemaphoreType.DMA,
    ],
)


NB = 50
NBLK = VOCAB // NB

def _final_body(h_ref, r1_ref, r2_ref, w1_ref, w2_ref, rho_ref, ln_ref, w_ref,
                o_ref, nrm_ref):
    @pl.when(pl.program_id(0) == 0)
    def _():
        h = _hop_update(h_ref[...], r1_ref[...], r2_ref[...],
                        w1_ref[...], w2_ref[...], rho_ref[...])
        mean = jnp.mean(h * h, axis=1, keepdims=True)
        nrm_ref[...] = (h * lax.rsqrt(mean + 1e-6) * ln_ref[...]).astype(BF)

    o_ref[...] = lax.dot_general(nrm_ref[...], w_ref[...], (((1,), (1,)), ((), ())),
                                 preferred_element_type=F32)


_final = pl.pallas_call(
    _final_body,
    grid=(NB,),
    in_specs=[
        pl.BlockSpec((T, D), lambda j: (0, 0)),
        pl.BlockSpec((T, D), lambda j: (0, 0)),
        pl.BlockSpec((T, D), lambda j: (0, 0)),
        pl.BlockSpec((T, 1), lambda j: (0, 0)),
        pl.BlockSpec((T, 1), lambda j: (0, 0)),
        pl.BlockSpec((T, 1), lambda j: (0, 0)),
        pl.BlockSpec((1, D), lambda j: (0, 0)),
        pl.BlockSpec((NBLK, D), lambda j: (j, 0)),
    ],
    out_specs=pl.BlockSpec((T, NBLK), lambda j: (0, j)),
    out_shape=jax.ShapeDtypeStruct((T, VOCAB), F32),
    scratch_shapes=[pltpu.VMEM((T, D), BF)],
)


def kernel(ids_t, embed_W, ln_scale, router_W, router_b, W1, b1, W2, b2):
    ids = ids_t.astype(I32)
    h = _embed_gather(embed_W, ids)
    rw = jnp.pad(jnp.transpose(router_W, (0, 2, 1)).astype(BF),
                 ((0, 0), (0, 0), (0, 128 - E)))
    rbp = jnp.pad(router_b, ((0, 0), (0, 128 - E)))[:, None, :]
    W1b = W1.astype(BF)
    W2b = W2.astype(BF)

    def hop_ffn(hop_idx, hcur, d1, d2, c1, c2):
        exp_in = _dispatch(hcur, d1.reshape(T), d2.reshape(T))
        eo = _ffn(exp_in.reshape(E, C, D), W1b[hop_idx],
                  b1[hop_idx].reshape(E, 1, F), W2b[hop_idx],
                  b2[hop_idx].reshape(E, 1, D))
        return _comb_gather(eo.reshape(EC, D), c1.reshape(T), c2.reshape(T))

    d1, d2, c1, c2, w1b, w2b, rho = _router0(h, rw[0], rbp[0])
    ra, rb2 = hop_ffn(0, h, d1, d2, c1, c2)
    d1, d2, c1, c2, w1b2, w2b2, rho2, hn = _router1(
        h, ra, rb2, w1b, w2b, rho, rw[1], rbp[1])
    ra, rb2 = hop_ffn(1, hn, d1, d2, c1, c2)
    return _final(hn, ra, rb2, w1b2, w2b2, rho2, ln_scale[None],
                  embed_W.astype(BF))


# speedup vs baseline: 2.4589x; 1.4306x over previous
import jax
import jax.numpy as jnp
from jax import lax
from jax.experimental import pallas as pl
from jax.experimental.pallas import tpu as pltpu
from jax.experimental.pallas import tpu_sc as plsc

VOCAB = 32000
D = 1024
E = 8
C = 640
F = 2048
T = 2048
NH = 2
EC = E * C

NC, NS, L = 2, 16, 16
NW = NC * NS
ROWS_PW = T // NW
SLOTS_PW = EC // NW
GCH = 40

BF = jnp.bfloat16
F32 = jnp.float32
I32 = jnp.int32

_mesh = plsc.VectorSubcoreMesh(core_axis_name="c", subcore_axis_name="s")
_sc_params = pltpu.CompilerParams(needs_layout_passes=False)


def _wid():
    return lax.axis_index("s") * NC + lax.axis_index("c")


def _embed_body(tab, ids, out, idx_v, rows_v, sem):
    base = _wid() * ROWS_PW
    pltpu.sync_copy(ids.at[pl.ds(base, ROWS_PW)], idx_v)
    pltpu.async_copy(tab.at[idx_v], rows_v, sem).wait()
    pltpu.sync_copy(rows_v, out.at[pl.ds(base, ROWS_PW)])


_embed_gather = pl.kernel(
    _embed_body,
    compiler_params=_sc_params,
    out_type=jax.ShapeDtypeStruct((T, D), F32),
    mesh=_mesh,
    scratch_types=[
        pltpu.VMEM((ROWS_PW,), I32),
        pltpu.VMEM((ROWS_PW, D), F32),
        pltpu.SemaphoreType.DMA,
    ],
)


def _router_math(h, rw, rb, d1r, d2r, c1r, c2r, w1r, w2r, rhor):
    lg = lax.dot_general(h.astype(BF), rw, (((1,), (0,)), ((), ())),
                         preferred_element_type=F32)
    lane = lax.broadcasted_iota(I32, (T, 128), 1)
    real = lane < E
    lg = jnp.where(real, lg + rb, -1e30)
    m = jnp.max(lg, axis=1, keepdims=True)
    ex = jnp.where(real, jnp.exp(lg - m), 0.0)
    p = ex / jnp.sum(ex, axis=1, keepdims=True)
    m1 = jnp.max(p, axis=1, keepdims=True)
    i1 = jnp.min(jnp.where(p == m1, lane, 128), axis=1, keepdims=True)
    s1 = lane == i1
    p_x = jnp.where(s1, -1.0, p)
    m2 = jnp.max(p_x, axis=1, keepdims=True)
    i2 = jnp.min(jnp.where(p_x == m2, lane, 128), axis=1, keepdims=True)
    s2 = lane == i2
    maskf = jnp.where(s1 | s2, 1.0, 0.0)
    cs = maskf
    sh = 1
    while sh < T:
        cs = cs + jnp.concatenate([jnp.zeros((sh, 128), F32), cs[:T - sh]], axis=0)
        sh *= 2
    pos = cs - 1.0
    p1 = jnp.sum(jnp.where(s1, pos, 0.0), axis=1, keepdims=True)
    p2 = jnp.sum(jnp.where(s2, pos, 0.0), axis=1, keepdims=True)
    w1 = jnp.sum(jnp.where(s1, p, 0.0), axis=1, keepdims=True)
    w2 = jnp.sum(jnp.where(s2, p, 0.0), axis=1, keepdims=True)
    k1 = p1 < C
    k2 = p2 < C
    e1f = i1.astype(F32)
    e2f = i2.astype(F32)
    d1r[...] = jnp.where(k1, e1f * C + p1, float(EC)).astype(I32)
    d2r[...] = jnp.where(k2, e2f * C + p2, float(EC)).astype(I32)
    c1r[...] = jnp.where(k1, e1f * C + p1, 0.0).astype(I32)
    c2r[...] = jnp.where(k2, e2f * C + p2, 0.0).astype(I32)
    w1o = jnp.where(k1, w1, 0.0)
    w2o = jnp.where(k2, w2, 0.0)
    w1r[...] = w1o.astype(BF).astype(F32)
    w2r[...] = w2o.astype(BF).astype(F32)
    rhor[...] = w1o + w2o


def _router0_body(h_ref, rw_ref, rb_ref,
                  d1r, d2r, c1r, c2r, w1r, w2r, rhor):
    _router_math(h_ref[...], rw_ref[...], rb_ref[...],
                 d1r, d2r, c1r, c2r, w1r, w2r, rhor)


def _hop_update(h, r1, r2, w1b, w2b, rho):
    return (h + (w1b * r1.astype(F32) + w2b * r2.astype(F32))) - rho * h


def _router1_body(h_ref, r1_ref, r2_ref, wp1_ref, wp2_ref, rhop_ref, rw_ref, rb_ref,
                  d1r, d2r, c1r, c2r, w1r, w2r, rhor, hn_r):
    h = _hop_update(h_ref[...], r1_ref[...], r2_ref[...],
                    wp1_ref[...], wp2_ref[...], rhop_ref[...])
    hn_r[...] = h
    _router_math(h, rw_ref[...], rb_ref[...],
                 d1r, d2r, c1r, c2r, w1r, w2r, rhor)


_ROUTER_OUTS = ([jax.ShapeDtypeStruct((T, 1), I32)] * 4
                + [jax.ShapeDtypeStruct((T, 1), F32)] * 3)

_router0 = pl.pallas_call(_router0_body, out_shape=_ROUTER_OUTS)
_router1 = pl.pallas_call(
    _router1_body,
    out_shape=_ROUTER_OUTS + [jax.ShapeDtypeStruct((T, D), F32)],
)


def _disp_body(h_hbm, d1_hbm, d2_hbm, out_hbm, d1_v, d2_v, s2t_v, ra_v, rb_v, sem):
    cp1 = pltpu.async_copy(d1_hbm, d1_v, sem)
    cp2 = pltpu.async_copy(d2_hbm, d2_v, sem)
    cp1.wait()
    cp2.wait()

    def z_body(i, _):
        s2t_v[pl.ds(i * L, L)] = jnp.zeros((L,), I32)
        return 0

    lax.fori_loop(0, (EC + L) // L, z_body, 0, unroll=4)

    def sc_body(i, _):
        vals = lax.iota(I32, L) + i * L
        i1 = d1_v[pl.ds(i * L, L)]
        i2 = d2_v[pl.ds(i * L, L)]
        plsc.store_scatter(s2t_v, [i1], vals, mask=i1 < EC)
        plsc.store_scatter(s2t_v, [i2], vals, mask=i2 < EC)
        return 0

    lax.fori_loop(0, T // L, sc_body, 0, unroll=4)
    seg = _wid() * SLOTS_PW
    bufs = [ra_v, rb_v]
    cps = [None, None]
    nch = SLOTS_PW // GCH
    for ch in range(nch):
        b = ch % 2
        if cps[b] is not None:
            cps[b].wait()
            pltpu.sync_copy(bufs[b], out_hbm.at[pl.ds(seg + (ch - 2) * GCH, GCH)])
        cps[b] = pltpu.async_copy(
            h_hbm.at[s2t_v.at[pl.ds(seg + ch * GCH, GCH)]], bufs[b], sem)
    for ch in (nch - 2, nch - 1):
        b = ch % 2
        cps[b].wait()
        pltpu.sync_copy(bufs[b], out_hbm.at[pl.ds(seg + ch * GCH, GCH)])


_dispatch = pl.kernel(
    _disp_body,
    compiler_params=_sc_params,
    out_type=jax.ShapeDtypeStruct((EC, D), F32),
    mesh=_mesh,
    scratch_types=[
        pltpu.VMEM((T,), I32),
        pltpu.VMEM((T,), I32),
        pltpu.VMEM((EC + L,), I32),
        pltpu.VMEM((GCH, D), F32),
        pltpu.VMEM((GCH, D), F32),
        pltpu.SemaphoreType.DMA,
    ],
)


def _ffn_body(x_ref, w1_ref, b1_ref, w2_ref, b2_ref, o_ref):
    h1 = lax.dot_general(x_ref[0].astype(BF), w1_ref[0, 0].astype(BF),
                         (((1,), (0,)), ((), ())), preferred_element_type=F32)
    h1 = jnp.maximum(h1 + b1_ref[0], 0.0)
    o = lax.dot_general(h1.astype(BF), w2_ref[0, 0].astype(BF),
                        (((1,), (0,)), ((), ())), preferred_element_type=F32)
    o_ref[0] = (o + b2_ref[0]).astype(BF).astype(F32)


def _make_ffn(hop):
    return pl.pallas_call(
        _ffn_body,
        grid=(E,),
        in_specs=[
            pl.BlockSpec((1, C, D), lambda e: (e, 0, 0)),
            pl.BlockSpec((1, 1, D, F), lambda e, h=hop: (h, e, 0, 0)),
            pl.BlockSpec((1, 1, F), lambda e: (e, 0, 0)),
            pl.BlockSpec((1, 1, F, D), lambda e, h=hop: (h, e, 0, 0)),
            pl.BlockSpec((1, 1, D), lambda e: (e, 0, 0)),
        ],
        out_specs=pl.BlockSpec((1, C, D), lambda e: (e, 0, 0)),
        out_shape=jax.ShapeDtypeStruct((E, C, D), F32),
    )


_ffn_hop = (_make_ffn(0), _make_ffn(1))


HGH = ROWS_PW // 2


def _gath_body(eo_hbm, c1_hbm, c2_hbm, r1_hbm, r2_hbm, c1_v, c2_v, ra_v, rb_v, sem):
    t0 = _wid() * ROWS_PW
    cp1 = pltpu.async_copy(c1_hbm.at[pl.ds(t0, ROWS_PW)], c1_v, sem)
    cp2 = pltpu.async_copy(c2_hbm.at[pl.ds(t0, ROWS_PW)], c2_v, sem)
    cp1.wait()
    cp2.wait()
    chunks = [(c1_v, r1_hbm, 0), (c2_v, r2_hbm, 0), (c1_v, r1_hbm, 1), (c2_v, r2_hbm, 1)]
    bufs = [ra_v, rb_v]
    cps = [None, None]
    for ch, (cv, out, half) in enumerate(chunks):
        b = ch % 2
        if cps[b] is not None:
            pcv, pout, phalf = chunks[ch - 2]
            cps[b].wait()
            pltpu.sync_copy(bufs[b], pout.at[pl.ds(t0 + phalf * HGH, HGH)])
        cps[b] = pltpu.async_copy(eo_hbm.at[cv.at[pl.ds(half * HGH, HGH)]], bufs[b], sem)
    for ch in (2, 3):
        b = ch % 2
        pcv, pout, phalf = chunks[ch]
        cps[b].wait()
        pltpu.sync_copy(bufs[b], pout.at[pl.ds(t0 + phalf * HGH, HGH)])


_comb_gather = pl.kernel(
    _gath_body,
    compiler_params=_sc_params,
    out_type=[jax.ShapeDtypeStruct((T, D), F32)] * 2,
    mesh=_mesh,
    scratch_types=[
        pltpu.VMEM((ROWS_PW,), I32),
        pltpu.VMEM((ROWS_PW,), I32),
        pltpu.VMEM((HGH, D), F32),
        pltpu.VMEM((HGH, D), F32),
        pltpu.SemaphoreType.DMA,
    ],
)


NB = 50
NBLK = VOCAB // NB

def _final_body(h_ref, r1_ref, r2_ref, w1_ref, w2_ref, rho_ref, ln_ref, w_ref,
                o_ref, nrm_ref):
    @pl.when(pl.program_id(0) == 0)
    def _():
        h = _hop_update(h_ref[...], r1_ref[...], r2_ref[...],
                        w1_ref[...], w2_ref[...], rho_ref[...])
        mean = jnp.mean(h * h, axis=1, keepdims=True)
        nrm_ref[...] = (h * lax.rsqrt(mean + 1e-6) * ln_ref[...]).astype(BF)

    o_ref[...] = lax.dot_general(nrm_ref[...], w_ref[...].astype(BF),
                                 (((1,), (1,)), ((), ())),
                                 preferred_element_type=F32)


_final = pl.pallas_call(
    _final_body,
    grid=(NB,),
    in_specs=[
        pl.BlockSpec((T, D), lambda j: (0, 0)),
        pl.BlockSpec((T, D), lambda j: (0, 0)),
        pl.BlockSpec((T, D), lambda j: (0, 0)),
        pl.BlockSpec((T, 1), lambda j: (0, 0)),
        pl.BlockSpec((T, 1), lambda j: (0, 0)),
        pl.BlockSpec((T, 1), lambda j: (0, 0)),
        pl.BlockSpec((1, D), lambda j: (0, 0)),
        pl.BlockSpec((NBLK, D), lambda j: (j, 0)),
    ],
    out_specs=pl.BlockSpec((T, NBLK), lambda j: (0, j)),
    out_shape=jax.ShapeDtypeStruct((T, VOCAB), F32),
    scratch_shapes=[pltpu.VMEM((T, D), BF)],
)


def kernel(ids_t, embed_W, ln_scale, router_W, router_b, W1, b1, W2, b2):
    ids = ids_t.astype(I32)
    h = _embed_gather(embed_W, ids)
    rw = jnp.pad(jnp.transpose(router_W, (0, 2, 1)).astype(BF),
                 ((0, 0), (0, 0), (0, 128 - E)))
    rbp = jnp.pad(router_b, ((0, 0), (0, 128 - E)))[:, None, :]

    def hop_ffn(hop_idx, hcur, d1, d2, c1, c2):
        exp_in = _dispatch(hcur, d1.reshape(T), d2.reshape(T))
        eo = _ffn_hop[hop_idx](exp_in.reshape(E, C, D), W1,
                               b1[hop_idx].reshape(E, 1, F), W2,
                               b2[hop_idx].reshape(E, 1, D))
        return _comb_gather(eo.reshape(EC, D), c1.reshape(T), c2.reshape(T))

    d1, d2, c1, c2, w1b, w2b, rho = _router0(h, rw[0], rbp[0])
    ra, rb2 = hop_ffn(0, h, d1, d2, c1, c2)
    d1, d2, c1, c2, w1b2, w2b2, rho2, hn = _router1(
        h, ra, rb2, w1b, w2b, rho, rw[1], rbp[1])
    ra, rb2 = hop_ffn(1, hn, d1, d2, c1, c2)
    return _final(hn, ra, rb2, w1b2, w2b2, rho2, ln_scale[None], embed_W)


# speedup vs baseline: 2.6882x; 1.0933x over previous
import jax
import jax.numpy as jnp
from jax import lax
from jax.experimental import pallas as pl
from jax.experimental.pallas import tpu as pltpu
from jax.experimental.pallas import tpu_sc as plsc

VOCAB = 32000
D = 1024
E = 8
C = 640
F = 2048
T = 2048
NH = 2
EC = E * C

NC, NS, L = 2, 16, 16
NW = NC * NS
ROWS_PW = T // NW
SLOTS_PW = EC // NW
GCH = 40

BF = jnp.bfloat16
F32 = jnp.float32
I32 = jnp.int32

_mesh = plsc.VectorSubcoreMesh(core_axis_name="c", subcore_axis_name="s")
_sc_params = pltpu.CompilerParams(needs_layout_passes=False)

D2 = D // 2
_HI = -65536


def _pack_bf16(x):
    a = x[:, :D2].astype(BF).astype(F32)
    b = x[:, D2:].astype(BF).astype(F32)
    ia = lax.shift_right_logical(lax.bitcast_convert_type(a, I32), 16)
    ib = lax.bitcast_convert_type(b, I32) & _HI
    return ia | ib


def _unpack_bf16(p):
    lo = lax.bitcast_convert_type(p << 16, F32)
    hi = lax.bitcast_convert_type(p & _HI, F32)
    return jnp.concatenate([lo, hi], axis=1)


def _wid():
    return lax.axis_index("s") * NC + lax.axis_index("c")


def _embed_body(tab, ids, out, idx_v, rows_v, sem):
    base = _wid() * ROWS_PW
    pltpu.sync_copy(ids.at[pl.ds(base, ROWS_PW)], idx_v)
    pltpu.async_copy(tab.at[idx_v], rows_v, sem).wait()
    pltpu.sync_copy(rows_v, out.at[pl.ds(base, ROWS_PW)])


_embed_gather = pl.kernel(
    _embed_body,
    compiler_params=_sc_params,
    out_type=jax.ShapeDtypeStruct((T, D), F32),
    mesh=_mesh,
    scratch_types=[
        pltpu.VMEM((ROWS_PW,), I32),
        pltpu.VMEM((ROWS_PW, D), F32),
        pltpu.SemaphoreType.DMA,
    ],
)


def _router_math(h, rw, rb, d1r, d2r, c1r, c2r, w1r, w2r, rhor):
    lg = lax.dot_general(h.astype(BF), rw, (((1,), (0,)), ((), ())),
                         preferred_element_type=F32)
    lane = lax.broadcasted_iota(I32, (T, 128), 1)
    real = lane < E
    lg = jnp.where(real, lg + rb, -1e30)
    m = jnp.max(lg, axis=1, keepdims=True)
    ex = jnp.where(real, jnp.exp(lg - m), 0.0)
    p = ex / jnp.sum(ex, axis=1, keepdims=True)
    m1 = jnp.max(p, axis=1, keepdims=True)
    i1 = jnp.min(jnp.where(p == m1, lane, 128), axis=1, keepdims=True)
    s1 = lane == i1
    p_x = jnp.where(s1, -1.0, p)
    m2 = jnp.max(p_x, axis=1, keepdims=True)
    i2 = jnp.min(jnp.where(p_x == m2, lane, 128), axis=1, keepdims=True)
    s2 = lane == i2
    maskf = jnp.where(s1 | s2, 1.0, 0.0)
    cs = maskf
    sh = 1
    while sh < T:
        cs = cs + jnp.concatenate([jnp.zeros((sh, 128), F32), cs[:T - sh]], axis=0)
        sh *= 2
    pos = cs - 1.0
    p1 = jnp.sum(jnp.where(s1, pos, 0.0), axis=1, keepdims=True)
    p2 = jnp.sum(jnp.where(s2, pos, 0.0), axis=1, keepdims=True)
    w1 = jnp.sum(jnp.where(s1, p, 0.0), axis=1, keepdims=True)
    w2 = jnp.sum(jnp.where(s2, p, 0.0), axis=1, keepdims=True)
    k1 = p1 < C
    k2 = p2 < C
    e1f = i1.astype(F32)
    e2f = i2.astype(F32)
    d1r[...] = jnp.where(k1, e1f * C + p1, float(EC)).astype(I32)
    d2r[...] = jnp.where(k2, e2f * C + p2, float(EC)).astype(I32)
    c1r[...] = jnp.where(k1, e1f * C + p1, 0.0).astype(I32)
    c2r[...] = jnp.where(k2, e2f * C + p2, 0.0).astype(I32)
    w1o = jnp.where(k1, w1, 0.0)
    w2o = jnp.where(k2, w2, 0.0)
    w1r[...] = w1o.astype(BF).astype(F32)
    w2r[...] = w2o.astype(BF).astype(F32)
    rhor[...] = w1o + w2o


def _router0_body(h_ref, rw_ref, rb_ref,
                  d1r, d2r, c1r, c2r, w1r, w2r, rhor, hp_r):
    hp_r[...] = _pack_bf16(h_ref[...])
    _router_math(h_ref[...], rw_ref[...], rb_ref[...],
                 d1r, d2r, c1r, c2r, w1r, w2r, rhor)


def _hop_update(h, r1, r2, w1b, w2b, rho):
    return (h + (w1b * r1 + w2b * r2)) - rho * h


def _router1_body(h_ref, r1_ref, r2_ref, wp1_ref, wp2_ref, rhop_ref, rw_ref, rb_ref,
                  d1r, d2r, c1r, c2r, w1r, w2r, rhor, hn_r, hp_r):
    h = _hop_update(h_ref[...], _unpack_bf16(r1_ref[...]), _unpack_bf16(r2_ref[...]),
                    wp1_ref[...], wp2_ref[...], rhop_ref[...])
    hn_r[...] = h
    hp_r[...] = _pack_bf16(h)
    _router_math(h, rw_ref[...], rb_ref[...],
                 d1r, d2r, c1r, c2r, w1r, w2r, rhor)


_ROUTER_OUTS = ([jax.ShapeDtypeStruct((T, 1), I32)] * 4
                + [jax.ShapeDtypeStruct((T, 1), F32)] * 3)
_PACKED_T = jax.ShapeDtypeStruct((T, D2), I32)

_router0 = pl.pallas_call(_router0_body, out_shape=_ROUTER_OUTS + [_PACKED_T])
_router1 = pl.pallas_call(
    _router1_body,
    out_shape=_ROUTER_OUTS + [jax.ShapeDtypeStruct((T, D), F32), _PACKED_T],
)


def _disp_body(h_hbm, d1_hbm, d2_hbm, out_hbm, d1_v, d2_v, s2t_v, ra_v, rb_v, sem):
    cp1 = pltpu.async_copy(d1_hbm, d1_v, sem)
    cp2 = pltpu.async_copy(d2_hbm, d2_v, sem)
    cp1.wait()
    cp2.wait()

    def z_body(i, _):
        s2t_v[pl.ds(i * L, L)] = jnp.zeros((L,), I32)
        return 0

    lax.fori_loop(0, (EC + L) // L, z_body, 0, unroll=4)

    def sc_body(i, _):
        vals = lax.iota(I32, L) + i * L
        i1 = d1_v[pl.ds(i * L, L)]
        i2 = d2_v[pl.ds(i * L, L)]
        plsc.store_scatter(s2t_v, [i1], vals, mask=i1 < EC)
        plsc.store_scatter(s2t_v, [i2], vals, mask=i2 < EC)
        return 0

    lax.fori_loop(0, T // L, sc_body, 0, unroll=4)
    seg = _wid() * SLOTS_PW
    bufs = [ra_v, rb_v]
    cps = [None, None]
    nch = SLOTS_PW // GCH
    for ch in range(nch):
        b = ch % 2
        if cps[b] is not None:
            cps[b].wait()
            pltpu.sync_copy(bufs[b], out_hbm.at[pl.ds(seg + (ch - 2) * GCH, GCH)])
        cps[b] = pltpu.async_copy(
            h_hbm.at[s2t_v.at[pl.ds(seg + ch * GCH, GCH)]], bufs[b], sem)
    for ch in (nch - 2, nch - 1):
        b = ch % 2
        cps[b].wait()
        pltpu.sync_copy(bufs[b], out_hbm.at[pl.ds(seg + ch * GCH, GCH)])


_dispatch = pl.kernel(
    _disp_body,
    compiler_params=_sc_params,
    out_type=jax.ShapeDtypeStruct((EC, D2), I32),
    mesh=_mesh,
    scratch_types=[
        pltpu.VMEM((T,), I32),
        pltpu.VMEM((T,), I32),
        pltpu.VMEM((EC + L,), I32),
        pltpu.VMEM((GCH, D2), I32),
        pltpu.VMEM((GCH, D2), I32),
        pltpu.SemaphoreType.DMA,
    ],
)


def _ffn_body(x_ref, w1_ref, b1_ref, w2_ref, b2_ref, o_ref):
    x = _unpack_bf16(x_ref[0]).astype(BF)
    h1 = lax.dot_general(x, w1_ref[0, 0].astype(BF),
                         (((1,), (0,)), ((), ())), preferred_element_type=F32)
    h1 = jnp.maximum(h1 + b1_ref[0], 0.0)
    o = lax.dot_general(h1.astype(BF), w2_ref[0, 0].astype(BF),
                        (((1,), (0,)), ((), ())), preferred_element_type=F32)
    o_ref[0] = _pack_bf16(o + b2_ref[0])


def _make_ffn(hop):
    return pl.pallas_call(
        _ffn_body,
        grid=(E,),
        in_specs=[
            pl.BlockSpec((1, C, D2), lambda e: (e, 0, 0)),
            pl.BlockSpec((1, 1, D, F), lambda e, h=hop: (h, e, 0, 0)),
            pl.BlockSpec((1, 1, F), lambda e: (e, 0, 0)),
            pl.BlockSpec((1, 1, F, D), lambda e, h=hop: (h, e, 0, 0)),
            pl.BlockSpec((1, 1, D), lambda e: (e, 0, 0)),
        ],
        out_specs=pl.BlockSpec((1, C, D2), lambda e: (e, 0, 0)),
        out_shape=jax.ShapeDtypeStruct((E, C, D2), I32),
    )


_ffn_hop = (_make_ffn(0), _make_ffn(1))


HGH = ROWS_PW // 2


def _gath_body(eo_hbm, c1_hbm, c2_hbm, r1_hbm, r2_hbm, c1_v, c2_v, ra_v, rb_v, sem):
    t0 = _wid() * ROWS_PW
    cp1 = pltpu.async_copy(c1_hbm.at[pl.ds(t0, ROWS_PW)], c1_v, sem)
    cp2 = pltpu.async_copy(c2_hbm.at[pl.ds(t0, ROWS_PW)], c2_v, sem)
    cp1.wait()
    cp2.wait()
    chunks = [(c1_v, r1_hbm, 0), (c2_v, r2_hbm, 0), (c1_v, r1_hbm, 1), (c2_v, r2_hbm, 1)]
    bufs = [ra_v, rb_v]
    cps = [None, None]
    for ch, (cv, out, half) in enumerate(chunks):
        b = ch % 2
        if cps[b] is not None:
            pcv, pout, phalf = chunks[ch - 2]
            cps[b].wait()
            pltpu.sync_copy(bufs[b], pout.at[pl.ds(t0 + phalf * HGH, HGH)])
        cps[b] = pltpu.async_copy(eo_hbm.at[cv.at[pl.ds(half * HGH, HGH)]], bufs[b], sem)
    for ch in (2, 3):
        b = ch % 2
        pcv, pout, phalf = chunks[ch]
        cps[b].wait()
        pltpu.sync_copy(bufs[b], pout.at[pl.ds(t0 + phalf * HGH, HGH)])


_comb_gather = pl.kernel(
    _gath_body,
    compiler_params=_sc_params,
    out_type=[jax.ShapeDtypeStruct((T, D2), I32)] * 2,
    mesh=_mesh,
    scratch_types=[
        pltpu.VMEM((ROWS_PW,), I32),
        pltpu.VMEM((ROWS_PW,), I32),
        pltpu.VMEM((HGH, D2), I32),
        pltpu.VMEM((HGH, D2), I32),
        pltpu.SemaphoreType.DMA,
    ],
)


NB = 50
NBLK = VOCAB // NB

def _final_body(h_ref, r1_ref, r2_ref, w1_ref, w2_ref, rho_ref, ln_ref, w_ref,
                o_ref, nrm_ref):
    @pl.when(pl.program_id(0) == 0)
    def _():
        h = _hop_update(h_ref[...], _unpack_bf16(r1_ref[...]),
                        _unpack_bf16(r2_ref[...]),
                        w1_ref[...], w2_ref[...], rho_ref[...])
        mean = jnp.mean(h * h, axis=1, keepdims=True)
        nrm_ref[...] = (h * lax.rsqrt(mean + 1e-6) * ln_ref[...]).astype(BF)

    o_ref[...] = lax.dot_general(nrm_ref[...], w_ref[...].astype(BF),
                                 (((1,), (1,)), ((), ())),
                                 preferred_element_type=F32)


_final = pl.pallas_call(
    _final_body,
    grid=(NB,),
    in_specs=[
        pl.BlockSpec((T, D), lambda j: (0, 0)),
        pl.BlockSpec((T, D2), lambda j: (0, 0)),
        pl.BlockSpec((T, D2), lambda j: (0, 0)),
        pl.BlockSpec((T, 1), lambda j: (0, 0)),
        pl.BlockSpec((T, 1), lambda j: (0, 0)),
        pl.BlockSpec((T, 1), lambda j: (0, 0)),
        pl.BlockSpec((1, D), lambda j: (0, 0)),
        pl.BlockSpec((NBLK, D), lambda j: (j, 0)),
    ],
    out_specs=pl.BlockSpec((T, NBLK), lambda j: (0, j)),
    out_shape=jax.ShapeDtypeStruct((T, VOCAB), F32),
    scratch_shapes=[pltpu.VMEM((T, D), BF)],
)


def kernel(ids_t, embed_W, ln_scale, router_W, router_b, W1, b1, W2, b2):
    ids = ids_t.astype(I32)
    h = _embed_gather(embed_W, ids)
    rw = jnp.pad(jnp.transpose(router_W, (0, 2, 1)).astype(BF),
                 ((0, 0), (0, 0), (0, 128 - E)))
    rbp = jnp.pad(router_b, ((0, 0), (0, 128 - E)))[:, None, :]

    def hop_ffn(hop_idx, hp, d1, d2, c1, c2):
        exp_in = _dispatch(hp, d1.reshape(T), d2.reshape(T))
        eo = _ffn_hop[hop_idx](exp_in.reshape(E, C, D2), W1,
                               b1[hop_idx].reshape(E, 1, F), W2,
                               b2[hop_idx].reshape(E, 1, D))
        return _comb_gather(eo.reshape(EC, D2), c1.reshape(T), c2.reshape(T))

    d1, d2, c1, c2, w1b, w2b, rho, hp0 = _router0(h, rw[0], rbp[0])
    ra, rb2 = hop_ffn(0, hp0, d1, d2, c1, c2)
    d1, d2, c1, c2, w1b2, w2b2, rho2, hn, hp1 = _router1(
        h, ra, rb2, w1b, w2b, rho, rw[1], rbp[1])
    ra, rb2 = hop_ffn(1, hp1, d1, d2, c1, c2)
    return _final(hn, ra, rb2, w1b2, w2b2, rho2, ln_scale[None], embed_W)


# speedup vs baseline: 2.6935x; 1.0020x over previous
import jax
import jax.numpy as jnp
from jax import lax
from jax.experimental import pallas as pl
from jax.experimental.pallas import tpu as pltpu
from jax.experimental.pallas import tpu_sc as plsc

VOCAB = 32000
D = 1024
E = 8
C = 640
F = 2048
T = 2048
NH = 2
EC = E * C

NC, NS, L = 2, 16, 16
NW = NC * NS
ROWS_PW = T // NW
SLOTS_PW = EC // NW
GCH = 40

BF = jnp.bfloat16
F32 = jnp.float32
I32 = jnp.int32

_mesh = plsc.VectorSubcoreMesh(core_axis_name="c", subcore_axis_name="s")
_sc_params = pltpu.CompilerParams(needs_layout_passes=False)

D2 = D // 2
_HI = -65536


def _pack_bf16(x):
    a = x[:, :D2].astype(BF).astype(F32)
    b = x[:, D2:].astype(BF).astype(F32)
    ia = lax.shift_right_logical(lax.bitcast_convert_type(a, I32), 16)
    ib = lax.bitcast_convert_type(b, I32) & _HI
    return ia | ib


def _unpack_bf16(p):
    lo = lax.bitcast_convert_type(p << 16, F32)
    hi = lax.bitcast_convert_type(p & _HI, F32)
    return jnp.concatenate([lo, hi], axis=1)


def _wid():
    return lax.axis_index("s") * NC + lax.axis_index("c")


def _embed_body(tab, ids, out, idx_v, rows_v, sem):
    base = _wid() * ROWS_PW
    pltpu.sync_copy(ids.at[pl.ds(base, ROWS_PW)], idx_v)
    pltpu.async_copy(tab.at[idx_v], rows_v, sem).wait()
    pltpu.sync_copy(rows_v, out.at[pl.ds(base, ROWS_PW)])


_embed_gather = pl.kernel(
    _embed_body,
    compiler_params=_sc_params,
    out_type=jax.ShapeDtypeStruct((T, D), F32),
    mesh=_mesh,
    scratch_types=[
        pltpu.VMEM((ROWS_PW,), I32),
        pltpu.VMEM((ROWS_PW, D), F32),
        pltpu.SemaphoreType.DMA,
    ],
)


def _router_math(h, rw, rb, d1r, d2r, c1r, c2r, w1r, w2r, rhor):
    lg = lax.dot_general(h.astype(BF), rw, (((1,), (0,)), ((), ())),
                         preferred_element_type=F32)
    lane = lax.broadcasted_iota(I32, (T, 128), 1)
    real = lane < E
    lg = jnp.where(real, lg + rb, -1e30)
    m = jnp.max(lg, axis=1, keepdims=True)
    ex = jnp.where(real, jnp.exp(lg - m), 0.0)
    p = ex / jnp.sum(ex, axis=1, keepdims=True)
    m1 = jnp.max(p, axis=1, keepdims=True)
    i1 = jnp.min(jnp.where(p == m1, lane, 128), axis=1, keepdims=True)
    s1 = lane == i1
    p_x = jnp.where(s1, -1.0, p)
    m2 = jnp.max(p_x, axis=1, keepdims=True)
    i2 = jnp.min(jnp.where(p_x == m2, lane, 128), axis=1, keepdims=True)
    s2 = lane == i2
    maskf = jnp.where(s1 | s2, 1.0, 0.0)
    cs = maskf
    sh = 1
    while sh < T:
        cs = cs + jnp.concatenate([jnp.zeros((sh, 128), F32), cs[:T - sh]], axis=0)
        sh *= 2
    pos = cs - 1.0
    p1 = jnp.sum(jnp.where(s1, pos, 0.0), axis=1, keepdims=True)
    p2 = jnp.sum(jnp.where(s2, pos, 0.0), axis=1, keepdims=True)
    w1 = jnp.sum(jnp.where(s1, p, 0.0), axis=1, keepdims=True)
    w2 = jnp.sum(jnp.where(s2, p, 0.0), axis=1, keepdims=True)
    k1 = p1 < C
    k2 = p2 < C
    e1f = i1.astype(F32)
    e2f = i2.astype(F32)
    d1r[...] = jnp.where(k1, e1f * C + p1, float(EC)).astype(I32)
    d2r[...] = jnp.where(k2, e2f * C + p2, float(EC)).astype(I32)
    c1r[...] = jnp.where(k1, e1f * C + p1, 0.0).astype(I32)
    c2r[...] = jnp.where(k2, e2f * C + p2, 0.0).astype(I32)
    w1o = jnp.where(k1, w1, 0.0)
    w2o = jnp.where(k2, w2, 0.0)
    w1r[...] = w1o.astype(BF).astype(F32)
    w2r[...] = w2o.astype(BF).astype(F32)
    rhor[...] = w1o + w2o


def _router0_body(h_ref, rw_ref, rb_ref,
                  d1r, d2r, c1r, c2r, w1r, w2r, rhor, hp_r):
    hp_r[...] = _pack_bf16(h_ref[...])
    _router_math(h_ref[...], rw_ref[...], rb_ref[...],
                 d1r, d2r, c1r, c2r, w1r, w2r, rhor)


def _hop_update(h, r1, r2, w1b, w2b, rho):
    return (h + (w1b * r1 + w2b * r2)) - rho * h


def _router1_body(h_ref, r1_ref, r2_ref, wp1_ref, wp2_ref, rhop_ref, rw_ref, rb_ref,
                  d1r, d2r, c1r, c2r, w1r, w2r, rhor, hn_r, hp_r):
    h = _hop_update(h_ref[...], _unpack_bf16(r1_ref[...]), _unpack_bf16(r2_ref[...]),
                    wp1_ref[...], wp2_ref[...], rhop_ref[...])
    hn_r[...] = h
    hp_r[...] = _pack_bf16(h)
    _router_math(h, rw_ref[...], rb_ref[...],
                 d1r, d2r, c1r, c2r, w1r, w2r, rhor)


_ROUTER_OUTS = ([jax.ShapeDtypeStruct((T, 1), I32)] * 4
                + [jax.ShapeDtypeStruct((T, 1), F32)] * 3)
_PACKED_T = jax.ShapeDtypeStruct((T, D2), I32)

_router0 = pl.pallas_call(_router0_body, out_shape=_ROUTER_OUTS + [_PACKED_T])
_router1 = pl.pallas_call(
    _router1_body,
    out_shape=_ROUTER_OUTS + [jax.ShapeDtypeStruct((T, D), F32), _PACKED_T],
)


def _disp_body(h_hbm, d1_hbm, d2_hbm, out_hbm, d1_v, d2_v, s2t_v, ra_v, rb_v, sem):
    cp1 = pltpu.async_copy(d1_hbm, d1_v, sem)
    cp2 = pltpu.async_copy(d2_hbm, d2_v, sem)
    cp1.wait()
    cp2.wait()
    seg = _wid() * SLOTS_PW

    def z_body(i, _):
        s2t_v[pl.ds(i * L, L)] = jnp.zeros((L,), I32)
        return 0

    lax.fori_loop(0, (SLOTS_PW + L) // L, z_body, 0, unroll=4)

    def sc_body(i, _):
        vals = lax.iota(I32, L) + i * L
        j1 = d1_v[pl.ds(i * L, L)] - seg
        j2 = d2_v[pl.ds(i * L, L)] - seg
        m1 = (j1 >= 0) & (j1 < SLOTS_PW)
        m2 = (j2 >= 0) & (j2 < SLOTS_PW)
        plsc.store_scatter(s2t_v, [jnp.where(m1, j1, 0)], vals, mask=m1)
        plsc.store_scatter(s2t_v, [jnp.where(m2, j2, 0)], vals, mask=m2)
        return 0

    lax.fori_loop(0, T // L, sc_body, 0, unroll=4)
    bufs = [ra_v, rb_v]
    cps = [None, None]
    nch = SLOTS_PW // GCH
    for ch in range(nch):
        b = ch % 2
        if cps[b] is not None:
            cps[b].wait()
            pltpu.sync_copy(bufs[b], out_hbm.at[pl.ds(seg + (ch - 2) * GCH, GCH)])
        cps[b] = pltpu.async_copy(
            h_hbm.at[s2t_v.at[pl.ds(ch * GCH, GCH)]], bufs[b], sem)
    for ch in (nch - 2, nch - 1):
        b = ch % 2
        cps[b].wait()
        pltpu.sync_copy(bufs[b], out_hbm.at[pl.ds(seg + ch * GCH, GCH)])


_dispatch = pl.kernel(
    _disp_body,
    compiler_params=_sc_params,
    out_type=jax.ShapeDtypeStruct((EC, D2), I32),
    mesh=_mesh,
    scratch_types=[
        pltpu.VMEM((T,), I32),
        pltpu.VMEM((T,), I32),
        pltpu.VMEM((SLOTS_PW + L,), I32),
        pltpu.VMEM((GCH, D2), I32),
        pltpu.VMEM((GCH, D2), I32),
        pltpu.SemaphoreType.DMA,
    ],
)


def _ffn_body(x_ref, w1_ref, b1_ref, w2_ref, b2_ref, o_ref):
    x = _unpack_bf16(x_ref[0]).astype(BF)
    h1 = lax.dot_general(x, w1_ref[0, 0].astype(BF),
                         (((1,), (0,)), ((), ())), preferred_element_type=F32)
    h1 = jnp.maximum(h1 + b1_ref[0], 0.0)
    o = lax.dot_general(h1.astype(BF), w2_ref[0, 0].astype(BF),
                        (((1,), (0,)), ((), ())), preferred_element_type=F32)
    o_ref[0] = _pack_bf16(o + b2_ref[0])


def _make_ffn(hop):
    return pl.pallas_call(
        _ffn_body,
        grid=(E,),
        in_specs=[
            pl.BlockSpec((1, C, D2), lambda e: (e, 0, 0)),
            pl.BlockSpec((1, 1, D, F), lambda e, h=hop: (h, e, 0, 0)),
            pl.BlockSpec((1, 1, F), lambda e: (e, 0, 0)),
            pl.BlockSpec((1, 1, F, D), lambda e, h=hop: (h, e, 0, 0)),
            pl.BlockSpec((1, 1, D), lambda e: (e, 0, 0)),
        ],
        out_specs=pl.BlockSpec((1, C, D2), lambda e: (e, 0, 0)),
        out_shape=jax.ShapeDtypeStruct((E, C, D2), I32),
    )


_ffn_hop = (_make_ffn(0), _make_ffn(1))


HGH = ROWS_PW // 2


def _gath_body(eo_hbm, c1_hbm, c2_hbm, r1_hbm, r2_hbm, c1_v, c2_v, ra_v, rb_v, sem):
    t0 = _wid() * ROWS_PW
    cp1 = pltpu.async_copy(c1_hbm.at[pl.ds(t0, ROWS_PW)], c1_v, sem)
    cp2 = pltpu.async_copy(c2_hbm.at[pl.ds(t0, ROWS_PW)], c2_v, sem)
    cp1.wait()
    cp2.wait()
    chunks = [(c1_v, r1_hbm, 0), (c2_v, r2_hbm, 0), (c1_v, r1_hbm, 1), (c2_v, r2_hbm, 1)]
    bufs = [ra_v, rb_v]
    cps = [None, None]
    for ch, (cv, out, half) in enumerate(chunks):
        b = ch % 2
        if cps[b] is not None:
            pcv, pout, phalf = chunks[ch - 2]
            cps[b].wait()
            pltpu.sync_copy(bufs[b], pout.at[pl.ds(t0 + phalf * HGH, HGH)])
        cps[b] = pltpu.async_copy(eo_hbm.at[cv.at[pl.ds(half * HGH, HGH)]], bufs[b], sem)
    for ch in (2, 3):
        b = ch % 2
        pcv, pout, phalf = chunks[ch]
        cps[b].wait()
        pltpu.sync_copy(bufs[b], pout.at[pl.ds(t0 + phalf * HGH, HGH)])


_comb_gather = pl.kernel(
    _gath_body,
    compiler_params=_sc_params,
    out_type=[jax.ShapeDtypeStruct((T, D2), I32)] * 2,
    mesh=_mesh,
    scratch_types=[
        pltpu.VMEM((ROWS_PW,), I32),
        pltpu.VMEM((ROWS_PW,), I32),
        pltpu.VMEM((HGH, D2), I32),
        pltpu.VMEM((HGH, D2), I32),
        pltpu.SemaphoreType.DMA,
    ],
)


NB = 50
NBLK = VOCAB // NB

def _final_body(h_ref, r1_ref, r2_ref, w1_ref, w2_ref, rho_ref, ln_ref, w_ref,
                o_ref, nrm_ref):
    @pl.when(pl.program_id(0) == 0)
    def _():
        h = _hop_update(h_ref[...], _unpack_bf16(r1_ref[...]),
                        _unpack_bf16(r2_ref[...]),
                        w1_ref[...], w2_ref[...], rho_ref[...])
        mean = jnp.mean(h * h, axis=1, keepdims=True)
        nrm_ref[...] = (h * lax.rsqrt(mean + 1e-6) * ln_ref[...]).astype(BF)

    o_ref[...] = lax.dot_general(nrm_ref[...], w_ref[...].astype(BF),
                                 (((1,), (1,)), ((), ())),
                                 preferred_element_type=F32)


_final = pl.pallas_call(
    _final_body,
    grid=(NB,),
    in_specs=[
        pl.BlockSpec((T, D), lambda j: (0, 0)),
        pl.BlockSpec((T, D2), lambda j: (0, 0)),
        pl.BlockSpec((T, D2), lambda j: (0, 0)),
        pl.BlockSpec((T, 1), lambda j: (0, 0)),
        pl.BlockSpec((T, 1), lambda j: (0, 0)),
        pl.BlockSpec((T, 1), lambda j: (0, 0)),
        pl.BlockSpec((1, D), lambda j: (0, 0)),
        pl.BlockSpec((NBLK, D), lambda j: (j, 0)),
    ],
    out_specs=pl.BlockSpec((T, NBLK), lambda j: (0, j)),
    out_shape=jax.ShapeDtypeStruct((T, VOCAB), F32),
    scratch_shapes=[pltpu.VMEM((T, D), BF)],
)


def kernel(ids_t, embed_W, ln_scale, router_W, router_b, W1, b1, W2, b2):
    ids = ids_t.astype(I32)
    h = _embed_gather(embed_W, ids)
    rw = jnp.pad(jnp.transpose(router_W, (0, 2, 1)).astype(BF),
                 ((0, 0), (0, 0), (0, 128 - E)))
    rbp = jnp.pad(router_b, ((0, 0), (0, 128 - E)))[:, None, :]

    def hop_ffn(hop_idx, hp, d1, d2, c1, c2):
        exp_in = _dispatch(hp, d1.reshape(T), d2.reshape(T))
        eo = _ffn_hop[hop_idx](exp_in.reshape(E, C, D2), W1,
                               b1[hop_idx].reshape(E, 1, F), W2,
                               b2[hop_idx].reshape(E, 1, D))
        return _comb_gather(eo.reshape(EC, D2), c1.reshape(T), c2.reshape(T))

    d1, d2, c1, c2, w1b, w2b, rho, hp0 = _router0(h, rw[0], rbp[0])
    ra, rb2 = hop_ffn(0, hp0, d1, d2, c1, c2)
    d1, d2, c1, c2, w1b2, w2b2, rho2, hn, hp1 = _router1(
        h, ra, rb2, w1b, w2b, rho, rw[1], rbp[1])
    ra, rb2 = hop_ffn(1, hp1, d1, d2, c1, c2)
    return _final(hn, ra, rb2, w1b2, w2b2, rho2, ln_scale[None], embed_W)
